# Initial kernel scaffold; baseline (speedup 1.0000x reference)
import functools

import jax
import jax.numpy as jnp
from jax import lax
from jax.experimental import pallas as pl
from jax.experimental.pallas import tpu as pltpu
from jax.experimental.pallas import tpu_sc as plsc

N = 100000
D_N = 64
D_C = 32
D_OUT = D_N + D_C
NW = 32
CHUNK = 128
K = 25
B_W = CHUNK * K
LAST_BASE = N - B_W


def kernel(node_type, charge_state, W_node_type, W_charge_state):
    mesh = plsc.VectorSubcoreMesh(core_axis_name="c", subcore_axis_name="s")

    @functools.partial(
        pl.kernel,
        mesh=mesh,
        out_type=jax.ShapeDtypeStruct((N, D_OUT), jnp.float32),
        scratch_types=[
            pltpu.VMEM((CHUNK,), jnp.int32),
            pltpu.VMEM((CHUNK,), jnp.int32),
            pltpu.VMEM((CHUNK, D_N), jnp.float32),
            pltpu.VMEM((CHUNK, D_C), jnp.float32),
            pltpu.SemaphoreType.DMA,
            pltpu.SemaphoreType.DMA,
        ],
    )
    def body(nt_hbm, cs_hbm, wn_hbm, wc_hbm, out_hbm,
             idxn_v, idxc_v, rn_v, rc_v, sem_n, sem_c):
        wid = lax.axis_index("s") * 2 + lax.axis_index("c")
        base = jnp.where(wid == NW - 1, LAST_BASE, wid * B_W)

        def chunk_body(j, carry):
            row0 = base + j * CHUNK
            pltpu.sync_copy(nt_hbm.at[pl.ds(row0, CHUNK)], idxn_v)
            pltpu.sync_copy(cs_hbm.at[pl.ds(row0, CHUNK)], idxc_v)
            cp_n = pltpu.async_copy(wn_hbm.at[idxn_v], rn_v, sem_n)
            cp_c = pltpu.async_copy(wc_hbm.at[idxc_v], rc_v, sem_c)
            cp_n.wait()
            cp_c.wait()
            pltpu.sync_copy(rn_v, out_hbm.at[pl.ds(row0, CHUNK), pl.ds(0, D_N)])
            pltpu.sync_copy(rc_v, out_hbm.at[pl.ds(row0, CHUNK), pl.ds(D_N, D_C)])
            return carry

        lax.fori_loop(0, K, chunk_body, 0)

    return body(node_type, charge_state, W_node_type, W_charge_state)


# speedup vs baseline: 1.7890x; 1.7890x over previous
import functools

import jax
import jax.numpy as jnp
from jax import lax
from jax.experimental import pallas as pl
from jax.experimental.pallas import tpu as pltpu
from jax.experimental.pallas import tpu_sc as plsc

N = 100000
D_N = 64
D_C = 32
D_OUT = D_N + D_C
ROW = 128
NW = 32
CHUNK = 128
K = 25
B_W = CHUNK * K
LAST_BASE = N - B_W


def kernel(node_type, charge_state, W_node_type, W_charge_state):
    wn_pad = jnp.pad(W_node_type, ((0, 0), (0, ROW - D_N)))
    wc_pad = jnp.pad(W_charge_state, ((0, 0), (D_N, ROW - D_OUT)))

    mesh = plsc.VectorSubcoreMesh(core_axis_name="c", subcore_axis_name="s")

    @functools.partial(
        pl.kernel,
        mesh=mesh,
        out_type=jax.ShapeDtypeStruct((N, ROW), jnp.float32),
        scratch_types=[
            pltpu.VMEM((CHUNK,), jnp.int32),
            pltpu.VMEM((CHUNK,), jnp.int32),
            pltpu.VMEM((CHUNK, ROW), jnp.float32),
            pltpu.SemaphoreType.DMA,
            pltpu.SemaphoreType.DMA,
        ],
    )
    def body(nt_hbm, cs_hbm, wn_hbm, wc_hbm, out_hbm,
             idxn_v, idxc_v, r_v, sem_n, sem_c):
        wid = lax.axis_index("s") * 2 + lax.axis_index("c")
        base = jnp.where(wid == NW - 1, LAST_BASE, wid * B_W)

        def chunk_body(j, carry):
            row0 = base + j * CHUNK
            pltpu.sync_copy(nt_hbm.at[pl.ds(row0, CHUNK)], idxn_v)
            pltpu.sync_copy(cs_hbm.at[pl.ds(row0, CHUNK)], idxc_v)
            pltpu.async_copy(wn_hbm.at[idxn_v], r_v, sem_n).wait()
            pltpu.async_copy(wc_hbm.at[idxc_v], r_v, sem_c, add=True).wait()
            pltpu.sync_copy(r_v, out_hbm.at[pl.ds(row0, CHUNK), :])
            return carry

        lax.fori_loop(0, K, chunk_body, 0)

    out128 = body(node_type, charge_state, wn_pad, wc_pad)
    return out128[:, :D_OUT]


# speedup vs baseline: 2.7374x; 1.5301x over previous
import functools

import jax
import jax.numpy as jnp
from jax import lax
from jax.experimental import pallas as pl
from jax.experimental.pallas import tpu as pltpu
from jax.experimental.pallas import tpu_sc as plsc

N = 100000
D_N = 64
D_C = 32
D_OUT = D_N + D_C
ROW = 128
V_C = 1000
NW = 32
CHUNK = 128
K = 25
NBUF = 5
B_W = CHUNK * K
LAST_BASE = N - B_W


def kernel(node_type, charge_state, W_node_type, W_charge_state):
    wn_pad = jnp.pad(W_node_type, ((0, 0), (0, ROW - D_N)))
    wc_pad = jnp.pad(W_charge_state, ((0, 0), (D_N, ROW - D_OUT)))

    mesh = plsc.VectorSubcoreMesh(core_axis_name="c", subcore_axis_name="s")

    @functools.partial(
        pl.kernel,
        mesh=mesh,
        out_type=jax.ShapeDtypeStruct((N, ROW), jnp.float32),
        scratch_types=[
            pltpu.VMEM((B_W,), jnp.int32),
            pltpu.VMEM((B_W,), jnp.int32),
            pltpu.VMEM_SHARED((V_C, ROW), jnp.float32),
        ]
        + [pltpu.VMEM((CHUNK, ROW), jnp.float32) for _ in range(NBUF)]
        + [pltpu.SemaphoreType.DMA] * (1 + 3 * NBUF),
    )
    def body(nt_hbm, cs_hbm, wn_hbm, wc_hbm, out_hbm,
             idxn_v, idxc_v, wc_sh, *rest):
        r_v = rest[:NBUF]
        sem_i = rest[NBUF]
        sem_n = rest[NBUF + 1:2 * NBUF + 1]
        sem_c = rest[2 * NBUF + 1:3 * NBUF + 1]
        sem_w = rest[3 * NBUF + 1:4 * NBUF + 1]

        cid = lax.axis_index("c")
        sid = lax.axis_index("s")
        wid = sid * 2 + cid
        base = jnp.where(wid == NW - 1, LAST_BASE, wid * B_W)

        cpn = pltpu.async_copy(nt_hbm.at[pl.ds(base, B_W)], idxn_v, sem_i)
        cpc = pltpu.async_copy(cs_hbm.at[pl.ds(base, B_W)], idxc_v, sem_i)

        @pl.when(sid == 0)
        def _():
            pltpu.sync_copy(wc_hbm, wc_sh)

        cpn.wait()
        cpc.wait()

        def gather(j, b):
            pltpu.async_copy(
                wn_hbm.at[idxn_v.at[pl.ds(j * CHUNK, CHUNK)]], r_v[b], sem_n[b])

        def charge_add(j, b):
            pltpu.async_copy(
                wc_sh.at[idxc_v.at[pl.ds(j * CHUNK, CHUNK)]], r_v[b],
                sem_c[b], add=True)

        def write(j, b):
            row0 = base + j * CHUNK
            pltpu.async_copy(r_v[b], out_hbm.at[pl.ds(row0, CHUNK)], sem_w[b])

        def drain(sem_b):
            pltpu.make_async_copy(
                wn_hbm.at[pl.ds(0, CHUNK)], r_v[0], sem_b).wait()

        gather(0, 0)
        gather(1, 1)
        gather(2, 2)
        plsc.subcore_barrier()

        for i in (0, 1):
            b = i % NBUF
            gather(i + 3, (b + 3) % NBUF)
            drain(sem_n[b])
            charge_add(i, b)
            drain(sem_c[b])
            write(i, b)

        def group(g, carry):
            for b2 in range(NBUF):
                i = 2 + g * NBUF + b2
                b = (2 + b2) % NBUF
                bg = (b + 3) % NBUF
                drain(sem_w[bg])
                gather(i + 3, bg)
                drain(sem_n[b])
                charge_add(i, b)
                drain(sem_c[b])
                write(i, b)
            return carry

        lax.fori_loop(0, 4, group, 0)

        for i in (22, 23, 24):
            b = i % NBUF
            drain(sem_n[b])
            charge_add(i, b)
            drain(sem_c[b])
            write(i, b)

        for b in range(NBUF):
            drain(sem_w[b])

    out128 = body(node_type, charge_state, wn_pad, wc_pad)
    return out128[:, :D_OUT]
